# baseline (speedup 1.0000x reference)
import functools

import jax
import jax.numpy as jnp
import numpy as np
from jax import lax
from jax.experimental import pallas as pl
from jax.experimental.pallas import tpu as pltpu
from jax.experimental.pallas import tpu_sc as plsc

B, N, M = 8, 4096, 1024
C_DST, C_SUB = 256, 512
OUT = 256
EPS = 1e-05
TN = 512
T = N // TN
ROWS = B * N
HIGH = jax.lax.Precision.HIGHEST

HB = B // 2
ROWS2 = HB * N
RT2 = ROWS2 // TN

NC, NS, L = 2, 16, 16
NW = NC * NS
QPW = ROWS2 // NW
CQ = 32
NCH = QPW // CQ
VPC = OUT // L

HALF = OUT // 2
PSCALE = 4096.0


def _proj_body(sub_x_ref, w1b_ref, p_ref):
    sx = sub_x_ref[0]
    p = lax.dot_general(sx.astype(jnp.bfloat16),
                        w1b_ref[...].astype(jnp.bfloat16),
                        (((0,), (1,)), ((), ())),
                        preferred_element_type=jnp.float32)
    pq = lax.convert_element_type(
        lax.clamp(0.0, jnp.round(p * PSCALE) + 32768.0, 65535.0), jnp.int32)
    p_ref[...] = lax.bitwise_or(pq[:, :HALF],
                                lax.shift_left(pq[:, HALF:], 16))


def _project_sub(sub_x, w1b):
    return pl.pallas_call(
        _proj_body,
        grid=(B,),
        in_specs=[
            pl.BlockSpec((1, C_SUB, M), lambda b: (b, 0, 0)),
            pl.BlockSpec((OUT, C_SUB), lambda b: (0, 0)),
        ],
        out_specs=pl.BlockSpec((M, HALF), lambda b: (b, 0)),
        out_shape=jax.ShapeDtypeStruct((B * M, HALF), jnp.int32),
    )(sub_x, w1b)


def _knn_body(b0, xyz_ref, sxyzt_ref, x_ref, w1a_ref, q_ref, i_ref, w_ref):
    b = pl.program_id(0) + b0
    q = xyz_ref[0]
    s = sxyzt_ref[0]
    qx, qy, qz = q[0:1, :], q[1:2, :], q[2:3, :]
    sx, sy, sz = s[:, 0:1], s[:, 1:2], s[:, 2:3]
    qq = qx * qx + qy * qy + qz * qz
    ss = sx * sx + sy * sy + sz * sz
    dot = lax.dot_general(s.astype(jnp.bfloat16),
                          q.astype(jnp.bfloat16),
                          (((1,), (0,)), ((), ())),
                          preferred_element_type=jnp.float32)
    d = qq + ss - 2.0 * dot
    d = jnp.maximum(d, 0.0)

    subl = lax.broadcasted_iota(jnp.int32, (M, TN), 0)
    mins, idxs = [], []
    for k in range(3):
        mn = jnp.min(d, axis=0, keepdims=True)
        eq = d == mn
        ix = jnp.min(jnp.where(eq, subl, M), axis=0, keepdims=True)
        mins.append(mn)
        idxs.append(ix)
        if k < 2:
            d = jnp.where(subl == ix, jnp.inf, d)

    r0 = 1.0 / (mins[0] + 1e-08)
    r1 = 1.0 / (mins[1] + 1e-08)
    r2 = 1.0 / (mins[2] + 1e-08)
    rs = r0 + r1 + r2
    w_ref[...] = jnp.concatenate([r0 / rs, r1 / rs, r2 / rs],
                                 axis=0)[:, None, :]
    base = b * M
    i_ref[...] = (jnp.concatenate(idxs, axis=0) + base)[:, None, :]

    xt = x_ref[0]
    q_ref[...] = lax.dot_general(xt.astype(jnp.bfloat16),
                                 w1a_ref[...].astype(jnp.bfloat16),
                                 (((0,), (1,)), ((), ())),
                                 preferred_element_type=jnp.float32
                                 ) - (32768.0 / PSCALE)


def _knn_and_skip(xyz, sub_xyzt, x, w1a, b0):
    return pl.pallas_call(
        functools.partial(_knn_body, b0),
        grid=(HB, T),
        in_specs=[
            pl.BlockSpec((1, 3, TN), lambda b, t: (b + b0, 0, t)),
            pl.BlockSpec((1, M, 3), lambda b, t: (b + b0, 0, 0)),
            pl.BlockSpec((1, C_DST, TN), lambda b, t: (b + b0, 0, t)),
            pl.BlockSpec((OUT, C_DST), lambda b, t: (0, 0)),
        ],
        out_specs=[
            pl.BlockSpec((TN, OUT), lambda b, t: (b * T + t, 0)),
            pl.BlockSpec((3, 1, TN), lambda b, t: (0, 0, b * T + t)),
            pl.BlockSpec((3, 1, TN), lambda b, t: (0, 0, b * T + t)),
        ],
        out_shape=[
            jax.ShapeDtypeStruct((ROWS2, OUT), jnp.float32),
            jax.ShapeDtypeStruct((3, 1, ROWS2), jnp.int32),
            jax.ShapeDtypeStruct((3, 1, ROWS2), jnp.float32),
        ],
    )(xyz, sub_xyzt, x, w1a)


def _sc_body(p_hbm, q_hbm, i_hbm, w_hbm,
             h1_hbm, st_hbm,
             i0_v, i1_v, i2_v, w0_v, w1_v, w2_v,
             r0a, r1a, r2a, qa, r0b, r1b, r2b, qb, o_a, o_b, st_v,
             sem_a, sem_b, sem_o):
    wid = lax.axis_index("s") * NC + lax.axis_index("c")
    qbase = wid * QPW

    pltpu.sync_copy(i_hbm.at[0, 0, pl.ds(qbase, QPW)], i0_v)
    pltpu.sync_copy(i_hbm.at[1, 0, pl.ds(qbase, QPW)], i1_v)
    pltpu.sync_copy(i_hbm.at[2, 0, pl.ds(qbase, QPW)], i2_v)
    pltpu.sync_copy(w_hbm.at[0, 0, pl.ds(qbase, QPW)], w0_v.at[pl.ds(0, QPW)])
    pltpu.sync_copy(w_hbm.at[1, 0, pl.ds(qbase, QPW)], w1_v.at[pl.ds(0, QPW)])
    pltpu.sync_copy(w_hbm.at[2, 0, pl.ds(qbase, QPW)], w2_v.at[pl.ds(0, QPW)])

    zero = jnp.zeros((L,), jnp.float32)
    for v in range(VPC):
        st_v[0, pl.ds(v * L, L)] = zero
        st_v[1, pl.ds(v * L, L)] = zero

    bufs = ((r0a, r1a, r2a, qa, sem_a), (r0b, r1b, r2b, qb, sem_b))

    def fire(cb, bset):
        r0x, r1x, r2x, qx, sem = bset
        pltpu.async_copy(p_hbm.at[i0_v.at[pl.ds(cb, CQ)]], r0x, sem)
        pltpu.async_copy(p_hbm.at[i1_v.at[pl.ds(cb, CQ)]], r1x, sem)
        pltpu.async_copy(p_hbm.at[i2_v.at[pl.ds(cb, CQ)]], r2x, sem)
        pltpu.async_copy(q_hbm.at[pl.ds(qbase + cb, CQ)], qx, sem)

    def wait4(bset):
        r0x, _, _, qx, sem = bset
        for _k in range(3):
            pltpu.make_async_copy(p_hbm.at[i0_v.at[pl.ds(0, CQ)]], r0x,
                                  sem).wait()
        pltpu.make_async_copy(q_hbm.at[pl.ds(qbase, CQ)], qx, sem).wait()

    def wait_out(ox):
        pltpu.make_async_copy(ox, h1_hbm.at[pl.ds(qbase, CQ)], sem_o).wait()

    def compute(cb, bset, ox):
        r0x, r1x, r2x, qx, _ = bset

        def one_q(qi, _):
            inv = 1.0 / PSCALE
            a0 = jnp.full((L,), w0_v[pl.ds(cb + qi, L)][0] * inv)
            a1 = jnp.full((L,), w1_v[pl.ds(cb + qi, L)][0] * inv)
            a2 = jnp.full((L,), w2_v[pl.ds(cb + qi, L)][0] * inv)
            sixteen = jnp.full((L,), jnp.int32(16))
            lomask = jnp.full((L,), jnp.int32(65535))

            def upk(u32):
                lo = lax.convert_element_type(
                    lax.bitwise_and(u32, lomask), jnp.float32)
                hi = lax.convert_element_type(
                    lax.shift_right_logical(u32, sixteen), jnp.float32)
                return lo, hi

            for v in range(HALF // L):
                s16 = pl.ds(v * L, L)
                p0l, p0h = upk(r0x[qi, s16])
                p1l, p1h = upk(r1x[qi, s16])
                p2l, p2h = upk(r2x[qi, s16])
                slh = pl.ds(HALF + v * L, L)
                acc_l = qx[qi, s16] + a0 * p0l + a1 * p1l + a2 * p2l
                acc_h = qx[qi, slh] + a0 * p0h + a1 * p1h + a2 * p2h
                ox[qi, s16] = acc_l
                ox[qi, slh] = acc_h
                plsc.addupdate(st_v.at[0, s16], acc_l)
                plsc.addupdate(st_v.at[1, s16], acc_l * acc_l)
                plsc.addupdate(st_v.at[0, slh], acc_h)
                plsc.addupdate(st_v.at[1, slh], acc_h * acc_h)
            return _

        lax.fori_loop(0, CQ, one_q, None, unroll=8)
        pltpu.async_copy(ox, h1_hbm.at[pl.ds(qbase + cb, CQ)], sem_o)

    fire(0, bufs[0])

    def pair(h, _):
        g0 = 2 * h
        fire((g0 + 1) * CQ, bufs[1])
        wait4(bufs[0])

        @pl.when(h > 0)
        def _drain_a():
            wait_out(o_a)

        compute(g0 * CQ, bufs[0], o_a)

        @pl.when(g0 + 2 < NCH)
        def _fire_next():
            fire((g0 + 2) * CQ, bufs[0])

        wait4(bufs[1])

        @pl.when(h > 0)
        def _drain_b():
            wait_out(o_b)

        compute((g0 + 1) * CQ, bufs[1], o_b)
        return _

    lax.fori_loop(0, NCH // 2, pair, None, unroll=False)
    wait_out(o_a)
    wait_out(o_b)
    pltpu.sync_copy(st_v, st_hbm.at[wid])


def _sc_interp(p_flat, q_flat, i_all, w_all):
    mesh = plsc.VectorSubcoreMesh(core_axis_name="c", subcore_axis_name="s")
    fn = pl.kernel(
        _sc_body,
        out_type=[
            jax.ShapeDtypeStruct((ROWS2, OUT), jnp.float32),
            jax.ShapeDtypeStruct((NW, 2, OUT), jnp.float32),
        ],
        mesh=mesh,
        scratch_types=[
            pltpu.VMEM((QPW,), jnp.int32),
            pltpu.VMEM((QPW,), jnp.int32),
            pltpu.VMEM((QPW,), jnp.int32),
            pltpu.VMEM((QPW + L,), jnp.float32),
            pltpu.VMEM((QPW + L,), jnp.float32),
            pltpu.VMEM((QPW + L,), jnp.float32),
            pltpu.VMEM((CQ, HALF), jnp.int32),
            pltpu.VMEM((CQ, HALF), jnp.int32),
            pltpu.VMEM((CQ, HALF), jnp.int32),
            pltpu.VMEM((CQ, OUT), jnp.float32),
            pltpu.VMEM((CQ, HALF), jnp.int32),
            pltpu.VMEM((CQ, HALF), jnp.int32),
            pltpu.VMEM((CQ, HALF), jnp.int32),
            pltpu.VMEM((CQ, OUT), jnp.float32),
            pltpu.VMEM((CQ, OUT), jnp.float32),
            pltpu.VMEM((CQ, OUT), jnp.float32),
            pltpu.VMEM((2, OUT), jnp.float32),
            pltpu.SemaphoreType.DMA,
            pltpu.SemaphoreType.DMA,
            pltpu.SemaphoreType.DMA,
        ],
    )
    return fn(p_flat, q_flat, i_all, w_all)


def _mid_body(h1a_ref, h1b_ref, sc_ref, sh_ref, w2t_ref, h2_ref, st_ref):
    r = pl.program_id(0)
    h = jnp.where(r < RT2, h1a_ref[...], h1b_ref[...])
    hn = jnp.maximum(h * sc_ref[...] + sh_ref[...], 0.0)
    h2 = lax.dot_general(hn.astype(jnp.bfloat16),
                         w2t_ref[...].astype(jnp.bfloat16),
                         (((1,), (0,)), ((), ())),
                         preferred_element_type=jnp.float32)
    s1 = jnp.sum(h2, axis=0, keepdims=True)
    s2 = jnp.sum(h2 * h2, axis=0, keepdims=True)
    st_ref[...] = jnp.concatenate([s1, s2], axis=0)[None]
    h2_ref[...] = jnp.transpose(h2.astype(jnp.bfloat16), (1, 0))


def _mid_layer(h1a, h1b, scale1, shift1, w2t):
    grid_r = ROWS // TN
    return pl.pallas_call(
        _mid_body,
        grid=(grid_r,),
        in_specs=[
            pl.BlockSpec((TN, OUT), lambda r: (jnp.minimum(r, RT2 - 1), 0)),
            pl.BlockSpec((TN, OUT), lambda r: (jnp.maximum(r - RT2, 0), 0)),
            pl.BlockSpec((1, OUT), lambda r: (0, 0)),
            pl.BlockSpec((1, OUT), lambda r: (0, 0)),
            pl.BlockSpec((OUT, OUT), lambda r: (0, 0)),
        ],
        out_specs=[
            pl.BlockSpec((OUT, TN), lambda r: (0, r)),
            pl.BlockSpec((1, 2, OUT), lambda r: (r, 0, 0)),
        ],
        out_shape=[
            jax.ShapeDtypeStruct((OUT, ROWS), jnp.bfloat16),
            jax.ShapeDtypeStruct((grid_r, 2, OUT), jnp.float32),
        ],
    )(h1a, h1b, scale1, shift1, w2t)


def _out_body(h2_ref, sc_ref, sh_ref, o_ref):
    h = h2_ref[...].astype(jnp.float32)
    y = jnp.maximum(h * sc_ref[...] + sh_ref[...], 0.0)
    o_ref[...] = y[None]


def _final_layer(h2, scale2, shift2):
    return pl.pallas_call(
        _out_body,
        grid=(B, T),
        in_specs=[
            pl.BlockSpec((OUT, TN), lambda b, t: (0, b * T + t)),
            pl.BlockSpec((OUT, 1), lambda b, t: (0, 0)),
            pl.BlockSpec((OUT, 1), lambda b, t: (0, 0)),
        ],
        out_specs=pl.BlockSpec((1, OUT, TN), lambda b, t: (b, 0, t)),
        out_shape=jax.ShapeDtypeStruct((B, OUT, N), jnp.float32),
    )(h2, jnp.transpose(scale2), jnp.transpose(shift2))


def _fold_stats(sums, sumsq, g, bb):
    mean = sums / float(ROWS)
    var = sumsq / float(ROWS) - mean * mean
    inv = g / jnp.sqrt(var + EPS)
    scale = inv.reshape(1, OUT)
    shift = (bb - mean * inv).reshape(1, OUT)
    return scale, shift


@jax.jit
def kernel(x, xyz, sub_x, sub_xyz, W1, g1, b1, W2, g2, b2):
    sub_xyzt = jnp.transpose(sub_xyz, (0, 2, 1))
    w1a = W1[:, :C_DST]
    w1b = W1[:, C_DST:]

    p_flat = _project_sub(sub_x, w1b)
    qa_flat, ia, wa = _knn_and_skip(xyz, sub_xyzt, x, w1a, 0)
    h1a, st1a = _sc_interp(p_flat, qa_flat, ia, wa)
    qb_flat, ib, wb = _knn_and_skip(xyz, sub_xyzt, x, w1a, HB)
    h1b, st1b = _sc_interp(p_flat, qb_flat, ib, wb)

    s1 = jnp.sum(st1a, axis=0) + jnp.sum(st1b, axis=0)
    scale1, shift1 = _fold_stats(s1[0], s1[1], g1, b1)

    h2, st2 = _mid_layer(h1a, h1b, scale1, shift1, jnp.transpose(W2))
    s2 = jnp.sum(st2, axis=0)
    scale2, shift2 = _fold_stats(s2[0], s2[1], g2, b2)

    return _final_layer(h2, scale2, shift2)


# speedup vs baseline: 1.2854x; 1.2854x over previous
import functools

import jax
import jax.numpy as jnp
import numpy as np
from jax import lax
from jax.experimental import pallas as pl
from jax.experimental.pallas import tpu as pltpu
from jax.experimental.pallas import tpu_sc as plsc

B, N, M = 8, 4096, 1024
C_DST, C_SUB = 256, 512
OUT = 256
EPS = 1e-05
TN = 512
T = N // TN
ROWS = B * N
HIGH = jax.lax.Precision.HIGHEST

HB = B // 2
ROWS2 = HB * N
RT2 = ROWS2 // TN

NC, NS, L = 2, 16, 16
NW = NC * NS
QPW = ROWS2 // NW
CQ = 32
NCH = QPW // CQ
VPC = OUT // L

HALF = OUT // 2
PSCALE = 4096.0


def _proj_body(sub_x_ref, w1b_ref, p_ref):
    sx = sub_x_ref[0]
    p = lax.dot_general(sx.astype(jnp.bfloat16),
                        w1b_ref[...].astype(jnp.bfloat16),
                        (((0,), (1,)), ((), ())),
                        preferred_element_type=jnp.float32)
    pq = lax.convert_element_type(
        lax.clamp(0.0, jnp.round(p * PSCALE) + 32768.0, 65535.0), jnp.int32)
    p_ref[...] = lax.bitwise_or(pq[:, :HALF],
                                lax.shift_left(pq[:, HALF:], 16))


def _project_sub(sub_x, w1b):
    return pl.pallas_call(
        _proj_body,
        grid=(B,),
        in_specs=[
            pl.BlockSpec((1, C_SUB, M), lambda b: (b, 0, 0)),
            pl.BlockSpec((OUT, C_SUB), lambda b: (0, 0)),
        ],
        out_specs=pl.BlockSpec((M, HALF), lambda b: (b, 0)),
        out_shape=jax.ShapeDtypeStruct((B * M, HALF), jnp.int32),
    )(sub_x, w1b)


def _knn_body(b0, xyz_ref, sxyzt_ref, x_ref, w1a_ref, q_ref, i_ref, w_ref):
    b = pl.program_id(0) + b0
    q = xyz_ref[0]
    s = sxyzt_ref[0]
    qx, qy, qz = q[0:1, :], q[1:2, :], q[2:3, :]
    sx, sy, sz = s[:, 0:1], s[:, 1:2], s[:, 2:3]
    qq = qx * qx + qy * qy + qz * qz
    ss = sx * sx + sy * sy + sz * sz
    dot = lax.dot_general(s.astype(jnp.bfloat16),
                          q.astype(jnp.bfloat16),
                          (((1,), (0,)), ((), ())),
                          preferred_element_type=jnp.float32)
    d = qq + ss - 2.0 * dot
    d = jnp.maximum(d, 0.0)

    subl = lax.broadcasted_iota(jnp.int32, (M, TN), 0)
    mins, idxs = [], []
    for k in range(3):
        mn = jnp.min(d, axis=0, keepdims=True)
        eq = d == mn
        ix = jnp.min(jnp.where(eq, subl, M), axis=0, keepdims=True)
        mins.append(mn)
        idxs.append(ix)
        if k < 2:
            d = jnp.where(subl == ix, jnp.inf, d)

    r0 = 1.0 / (mins[0] + 1e-08)
    r1 = 1.0 / (mins[1] + 1e-08)
    r2 = 1.0 / (mins[2] + 1e-08)
    rs = r0 + r1 + r2
    w_ref[...] = jnp.concatenate([r0 / rs, r1 / rs, r2 / rs],
                                 axis=0)[:, None, :]
    base = b * M
    i_ref[...] = (jnp.concatenate(idxs, axis=0) + base)[:, None, :]

    xt = x_ref[0]
    q_ref[...] = lax.dot_general(xt.astype(jnp.bfloat16),
                                 w1a_ref[...].astype(jnp.bfloat16),
                                 (((0,), (1,)), ((), ())),
                                 preferred_element_type=jnp.float32
                                 ) - (32768.0 / PSCALE)


def _knn_and_skip(xyz, sub_xyzt, x, w1a, b0):
    return pl.pallas_call(
        functools.partial(_knn_body, b0),
        grid=(HB, T),
        in_specs=[
            pl.BlockSpec((1, 3, TN), lambda b, t: (b + b0, 0, t)),
            pl.BlockSpec((1, M, 3), lambda b, t: (b + b0, 0, 0)),
            pl.BlockSpec((1, C_DST, TN), lambda b, t: (b + b0, 0, t)),
            pl.BlockSpec((OUT, C_DST), lambda b, t: (0, 0)),
        ],
        out_specs=[
            pl.BlockSpec((TN, OUT), lambda b, t: (b * T + t, 0)),
            pl.BlockSpec((3, 1, TN), lambda b, t: (0, 0, b * T + t)),
            pl.BlockSpec((3, 1, TN), lambda b, t: (0, 0, b * T + t)),
        ],
        out_shape=[
            jax.ShapeDtypeStruct((ROWS2, OUT), jnp.float32),
            jax.ShapeDtypeStruct((3, 1, ROWS2), jnp.int32),
            jax.ShapeDtypeStruct((3, 1, ROWS2), jnp.float32),
        ],
    )(xyz, sub_xyzt, x, w1a)


def _sc_body(p_hbm, q_hbm, i_hbm, w_hbm,
             h1_hbm, st_hbm,
             i0_v, i1_v, i2_v, w0_v, w1_v, w2_v,
             r0a, r1a, r2a, qa, r0b, r1b, r2b, qb, o_a, o_b, st_v,
             sem_a, sem_b, sem_o):
    wid = lax.axis_index("s") * NC + lax.axis_index("c")
    qbase = wid * QPW

    pltpu.sync_copy(i_hbm.at[0, 0, pl.ds(qbase, QPW)], i0_v)
    pltpu.sync_copy(i_hbm.at[1, 0, pl.ds(qbase, QPW)], i1_v)
    pltpu.sync_copy(i_hbm.at[2, 0, pl.ds(qbase, QPW)], i2_v)
    pltpu.sync_copy(w_hbm.at[0, 0, pl.ds(qbase, QPW)], w0_v.at[pl.ds(0, QPW)])
    pltpu.sync_copy(w_hbm.at[1, 0, pl.ds(qbase, QPW)], w1_v.at[pl.ds(0, QPW)])
    pltpu.sync_copy(w_hbm.at[2, 0, pl.ds(qbase, QPW)], w2_v.at[pl.ds(0, QPW)])

    zero = jnp.zeros((L,), jnp.float32)
    for v in range(VPC):
        st_v[0, pl.ds(v * L, L)] = zero
        st_v[1, pl.ds(v * L, L)] = zero

    bufs = ((r0a, r1a, r2a, qa, sem_a), (r0b, r1b, r2b, qb, sem_b))

    def fire(cb, bset):
        r0x, r1x, r2x, qx, sem = bset
        pltpu.async_copy(p_hbm.at[i0_v.at[pl.ds(cb, CQ)]], r0x, sem)
        pltpu.async_copy(p_hbm.at[i1_v.at[pl.ds(cb, CQ)]], r1x, sem)
        pltpu.async_copy(p_hbm.at[i2_v.at[pl.ds(cb, CQ)]], r2x, sem)
        pltpu.async_copy(q_hbm.at[pl.ds(qbase + cb, CQ)], qx, sem)

    def wait4(bset):
        r0x, _, _, qx, sem = bset
        for _k in range(3):
            pltpu.make_async_copy(p_hbm.at[i0_v.at[pl.ds(0, CQ)]], r0x,
                                  sem).wait()
        pltpu.make_async_copy(q_hbm.at[pl.ds(qbase, CQ)], qx, sem).wait()

    def wait_out(ox):
        pltpu.make_async_copy(ox, h1_hbm.at[pl.ds(qbase, CQ)], sem_o).wait()

    def compute(cb, bset, ox):
        r0x, r1x, r2x, qx, _ = bset

        def one_q(qi, _):
            inv = 1.0 / PSCALE
            a0 = jnp.full((L,), w0_v[pl.ds(cb + qi, L)][0] * inv)
            a1 = jnp.full((L,), w1_v[pl.ds(cb + qi, L)][0] * inv)
            a2 = jnp.full((L,), w2_v[pl.ds(cb + qi, L)][0] * inv)
            sixteen = jnp.full((L,), jnp.int32(16))
            lomask = jnp.full((L,), jnp.int32(65535))

            def upk(u32):
                lo = lax.convert_element_type(
                    lax.bitwise_and(u32, lomask), jnp.float32)
                hi = lax.convert_element_type(
                    lax.shift_right_logical(u32, sixteen), jnp.float32)
                return lo, hi

            for v in range(HALF // L):
                s16 = pl.ds(v * L, L)
                p0l, p0h = upk(r0x[qi, s16])
                p1l, p1h = upk(r1x[qi, s16])
                p2l, p2h = upk(r2x[qi, s16])
                slh = pl.ds(HALF + v * L, L)
                acc_l = qx[qi, s16] + a0 * p0l + a1 * p1l + a2 * p2l
                acc_h = qx[qi, slh] + a0 * p0h + a1 * p1h + a2 * p2h
                ox[qi, s16] = acc_l
                ox[qi, slh] = acc_h
                plsc.addupdate(st_v.at[0, s16], acc_l)
                plsc.addupdate(st_v.at[1, s16], acc_l * acc_l)
                plsc.addupdate(st_v.at[0, slh], acc_h)
                plsc.addupdate(st_v.at[1, slh], acc_h * acc_h)
            return _

        lax.fori_loop(0, CQ, one_q, None, unroll=4)
        pltpu.async_copy(ox, h1_hbm.at[pl.ds(qbase + cb, CQ)], sem_o)

    fire(0, bufs[0])

    def pair(h, _):
        g0 = 2 * h
        fire((g0 + 1) * CQ, bufs[1])
        wait4(bufs[0])

        @pl.when(h > 0)
        def _drain_a():
            wait_out(o_a)

        compute(g0 * CQ, bufs[0], o_a)

        @pl.when(g0 + 2 < NCH)
        def _fire_next():
            fire((g0 + 2) * CQ, bufs[0])

        wait4(bufs[1])

        @pl.when(h > 0)
        def _drain_b():
            wait_out(o_b)

        compute((g0 + 1) * CQ, bufs[1], o_b)
        return _

    lax.fori_loop(0, NCH // 2, pair, None, unroll=False)
    wait_out(o_a)
    wait_out(o_b)
    pltpu.sync_copy(st_v, st_hbm.at[wid])


def _sc_interp(p_flat, q_flat, i_all, w_all):
    mesh = plsc.VectorSubcoreMesh(core_axis_name="c", subcore_axis_name="s")
    fn = pl.kernel(
        _sc_body,
        out_type=[
            jax.ShapeDtypeStruct((ROWS2, OUT), jnp.float32),
            jax.ShapeDtypeStruct((NW, 2, OUT), jnp.float32),
        ],
        mesh=mesh,
        scratch_types=[
            pltpu.VMEM((QPW,), jnp.int32),
            pltpu.VMEM((QPW,), jnp.int32),
            pltpu.VMEM((QPW,), jnp.int32),
            pltpu.VMEM((QPW + L,), jnp.float32),
            pltpu.VMEM((QPW + L,), jnp.float32),
            pltpu.VMEM((QPW + L,), jnp.float32),
            pltpu.VMEM((CQ, HALF), jnp.int32),
            pltpu.VMEM((CQ, HALF), jnp.int32),
            pltpu.VMEM((CQ, HALF), jnp.int32),
            pltpu.VMEM((CQ, OUT), jnp.float32),
            pltpu.VMEM((CQ, HALF), jnp.int32),
            pltpu.VMEM((CQ, HALF), jnp.int32),
            pltpu.VMEM((CQ, HALF), jnp.int32),
            pltpu.VMEM((CQ, OUT), jnp.float32),
            pltpu.VMEM((CQ, OUT), jnp.float32),
            pltpu.VMEM((CQ, OUT), jnp.float32),
            pltpu.VMEM((2, OUT), jnp.float32),
            pltpu.SemaphoreType.DMA,
            pltpu.SemaphoreType.DMA,
            pltpu.SemaphoreType.DMA,
        ],
    )
    return fn(p_flat, q_flat, i_all, w_all)


def _mid_body(h1a_ref, h1b_ref, sc_ref, sh_ref, w2t_ref, h2_ref, st_ref):
    r = pl.program_id(0)
    h = jnp.where(r < RT2, h1a_ref[...], h1b_ref[...])
    hn = jnp.maximum(h * sc_ref[...] + sh_ref[...], 0.0)
    h2 = lax.dot_general(hn.astype(jnp.bfloat16),
                         w2t_ref[...].astype(jnp.bfloat16),
                         (((1,), (0,)), ((), ())),
                         preferred_element_type=jnp.float32)
    s1 = jnp.sum(h2, axis=0, keepdims=True)
    s2 = jnp.sum(h2 * h2, axis=0, keepdims=True)
    st_ref[...] = jnp.concatenate([s1, s2], axis=0)[None]
    h2_ref[...] = h2.astype(jnp.bfloat16)


def _mid_layer(h1a, h1b, scale1, shift1, w2t):
    grid_r = ROWS // TN
    return pl.pallas_call(
        _mid_body,
        grid=(grid_r,),
        in_specs=[
            pl.BlockSpec((TN, OUT), lambda r: (jnp.minimum(r, RT2 - 1), 0)),
            pl.BlockSpec((TN, OUT), lambda r: (jnp.maximum(r - RT2, 0), 0)),
            pl.BlockSpec((1, OUT), lambda r: (0, 0)),
            pl.BlockSpec((1, OUT), lambda r: (0, 0)),
            pl.BlockSpec((OUT, OUT), lambda r: (0, 0)),
        ],
        out_specs=[
            pl.BlockSpec((TN, OUT), lambda r: (r, 0)),
            pl.BlockSpec((1, 2, OUT), lambda r: (r, 0, 0)),
        ],
        out_shape=[
            jax.ShapeDtypeStruct((ROWS, OUT), jnp.bfloat16),
            jax.ShapeDtypeStruct((grid_r, 2, OUT), jnp.float32),
        ],
    )(h1a, h1b, scale1, shift1, w2t)


def _out_body(h2_ref, sc_ref, sh_ref, o_ref):
    h = h2_ref[...].astype(jnp.float32)
    y = jnp.maximum(h * sc_ref[...] + sh_ref[...], 0.0)
    o_ref[...] = jnp.transpose(y, (1, 0))[None]


def _final_layer(h2, scale2, shift2):
    return pl.pallas_call(
        _out_body,
        grid=(B, T),
        in_specs=[
            pl.BlockSpec((TN, OUT), lambda b, t: (b * T + t, 0)),
            pl.BlockSpec((1, OUT), lambda b, t: (0, 0)),
            pl.BlockSpec((1, OUT), lambda b, t: (0, 0)),
        ],
        out_specs=pl.BlockSpec((1, OUT, TN), lambda b, t: (b, 0, t)),
        out_shape=jax.ShapeDtypeStruct((B, OUT, N), jnp.float32),
    )(h2, scale2, shift2)


def _fold_stats(sums, sumsq, g, bb):
    mean = sums / float(ROWS)
    var = sumsq / float(ROWS) - mean * mean
    inv = g / jnp.sqrt(var + EPS)
    scale = inv.reshape(1, OUT)
    shift = (bb - mean * inv).reshape(1, OUT)
    return scale, shift


@jax.jit
def kernel(x, xyz, sub_x, sub_xyz, W1, g1, b1, W2, g2, b2):
    sub_xyzt = jnp.transpose(sub_xyz, (0, 2, 1))
    w1a = W1[:, :C_DST]
    w1b = W1[:, C_DST:]

    p_flat = _project_sub(sub_x, w1b)
    qa_flat, ia, wa = _knn_and_skip(xyz, sub_xyzt, x, w1a, 0)
    h1a, st1a = _sc_interp(p_flat, qa_flat, ia, wa)
    qb_flat, ib, wb = _knn_and_skip(xyz, sub_xyzt, x, w1a, HB)
    h1b, st1b = _sc_interp(p_flat, qb_flat, ib, wb)

    s1 = jnp.sum(st1a, axis=0) + jnp.sum(st1b, axis=0)
    scale1, shift1 = _fold_stats(s1[0], s1[1], g1, b1)

    h2, st2 = _mid_layer(h1a, h1b, scale1, shift1, jnp.transpose(W2))
    s2 = jnp.sum(st2, axis=0)
    scale2, shift2 = _fold_stats(s2[0], s2[1], g2, b2)

    return _final_layer(h2, scale2, shift2)


# speedup vs baseline: 1.2855x; 1.0001x over previous
import functools

import jax
import jax.numpy as jnp
import numpy as np
from jax import lax
from jax.experimental import pallas as pl
from jax.experimental.pallas import tpu as pltpu
from jax.experimental.pallas import tpu_sc as plsc

B, N, M = 8, 4096, 1024
C_DST, C_SUB = 256, 512
OUT = 256
EPS = 1e-05
TN = 512
T = N // TN
ROWS = B * N

HB = B // 2
ROWS2 = HB * N
RT2 = ROWS2 // TN

NC, NS, L = 2, 16, 16
NW = NC * NS
QPW = ROWS2 // NW
CQ = 32
NCH = QPW // CQ
VPC = OUT // L

HALF = OUT // 2
PSCALE = 4096.0


def _proj_body(sub_x_ref, w1b_ref, p_ref):
    sx = sub_x_ref[0]
    p = lax.dot_general(sx.astype(jnp.bfloat16),
                        w1b_ref[...].astype(jnp.bfloat16),
                        (((0,), (1,)), ((), ())),
                        preferred_element_type=jnp.float32)
    pq = lax.convert_element_type(
        lax.clamp(0.0, jnp.round(p * PSCALE) + 32768.0, 65535.0), jnp.int32)
    p_ref[...] = lax.bitwise_or(pq[:, :HALF],
                                lax.shift_left(pq[:, HALF:], 16))


def _project_sub(sub_x, w1b):
    return pl.pallas_call(
        _proj_body,
        grid=(B,),
        in_specs=[
            pl.BlockSpec((1, C_SUB, M), lambda b: (b, 0, 0)),
            pl.BlockSpec((OUT, C_SUB), lambda b: (0, 0)),
        ],
        out_specs=pl.BlockSpec((M, HALF), lambda b: (b, 0)),
        out_shape=jax.ShapeDtypeStruct((B * M, HALF), jnp.int32),
    )(sub_x, w1b)


def _knn_body(b0, xyz_ref, sxyzt_ref, x_ref, w1a_ref, q_ref, i_ref, w_ref):
    b = pl.program_id(0) + b0
    q = xyz_ref[0]
    s = sxyzt_ref[0]
    qx, qy, qz = q[0:1, :], q[1:2, :], q[2:3, :]
    sx, sy, sz = s[:, 0:1], s[:, 1:2], s[:, 2:3]
    qq = qx * qx + qy * qy + qz * qz
    ss = sx * sx + sy * sy + sz * sz
    dot = lax.dot_general(s.astype(jnp.bfloat16),
                          q.astype(jnp.bfloat16),
                          (((1,), (0,)), ((), ())),
                          preferred_element_type=jnp.float32)
    d = qq + ss - 2.0 * dot
    d = jnp.maximum(d, 0.0)

    subl = lax.broadcasted_iota(jnp.int32, (M, TN), 0)
    mins, idxs = [], []
    for k in range(3):
        mn = jnp.min(d, axis=0, keepdims=True)
        eq = d == mn
        ix = jnp.min(jnp.where(eq, subl, M), axis=0, keepdims=True)
        mins.append(mn)
        idxs.append(ix)
        if k < 2:
            d = jnp.where(subl == ix, jnp.inf, d)

    r0 = 1.0 / (mins[0] + 1e-08)
    r1 = 1.0 / (mins[1] + 1e-08)
    r2 = 1.0 / (mins[2] + 1e-08)
    rs = r0 + r1 + r2
    w_ref[...] = jnp.concatenate([r0 / rs, r1 / rs, r2 / rs],
                                 axis=0)[:, None, :]
    base = b * M
    i_ref[...] = (jnp.concatenate(idxs, axis=0) + base)[:, None, :]

    xt = x_ref[0]
    q_ref[...] = lax.dot_general(xt.astype(jnp.bfloat16),
                                 w1a_ref[...].astype(jnp.bfloat16),
                                 (((0,), (1,)), ((), ())),
                                 preferred_element_type=jnp.float32
                                 ) - (32768.0 / PSCALE)


def _knn_and_skip(xyz, sub_xyzt, x, w1a, b0):
    return pl.pallas_call(
        functools.partial(_knn_body, b0),
        grid=(HB, T),
        in_specs=[
            pl.BlockSpec((1, 3, TN), lambda b, t: (b + b0, 0, t)),
            pl.BlockSpec((1, M, 3), lambda b, t: (b + b0, 0, 0)),
            pl.BlockSpec((1, C_DST, TN), lambda b, t: (b + b0, 0, t)),
            pl.BlockSpec((OUT, C_DST), lambda b, t: (0, 0)),
        ],
        out_specs=[
            pl.BlockSpec((TN, OUT), lambda b, t: (b * T + t, 0)),
            pl.BlockSpec((3, 1, TN), lambda b, t: (0, 0, b * T + t)),
            pl.BlockSpec((3, 1, TN), lambda b, t: (0, 0, b * T + t)),
        ],
        out_shape=[
            jax.ShapeDtypeStruct((ROWS2, OUT), jnp.float32),
            jax.ShapeDtypeStruct((3, 1, ROWS2), jnp.int32),
            jax.ShapeDtypeStruct((3, 1, ROWS2), jnp.float32),
        ],
    )(xyz, sub_xyzt, x, w1a)


def _sc_body(p_hbm, q_hbm, i_hbm, w_hbm,
             h1_hbm, st_hbm,
             i0_v, i1_v, i2_v, w0_v, w1_v, w2_v,
             r0a, r1a, r2a, qa, r0b, r1b, r2b, qb, o_a, o_b, st_v,
             sem_a, sem_b, sem_o):
    wid = lax.axis_index("s") * NC + lax.axis_index("c")
    qbase = wid * QPW

    pltpu.sync_copy(i_hbm.at[0, 0, pl.ds(qbase, QPW)], i0_v)
    pltpu.sync_copy(i_hbm.at[1, 0, pl.ds(qbase, QPW)], i1_v)
    pltpu.sync_copy(i_hbm.at[2, 0, pl.ds(qbase, QPW)], i2_v)
    pltpu.sync_copy(w_hbm.at[0, 0, pl.ds(qbase, QPW)], w0_v.at[pl.ds(0, QPW)])
    pltpu.sync_copy(w_hbm.at[1, 0, pl.ds(qbase, QPW)], w1_v.at[pl.ds(0, QPW)])
    pltpu.sync_copy(w_hbm.at[2, 0, pl.ds(qbase, QPW)], w2_v.at[pl.ds(0, QPW)])

    zero = jnp.zeros((L,), jnp.float32)
    for v in range(VPC):
        st_v[0, pl.ds(v * L, L)] = zero
        st_v[1, pl.ds(v * L, L)] = zero

    bufs = ((r0a, r1a, r2a, qa, sem_a), (r0b, r1b, r2b, qb, sem_b))

    def fire(cb, bset):
        r0x, r1x, r2x, qx, sem = bset
        pltpu.async_copy(p_hbm.at[i0_v.at[pl.ds(cb, CQ)]], r0x, sem)
        pltpu.async_copy(p_hbm.at[i1_v.at[pl.ds(cb, CQ)]], r1x, sem)
        pltpu.async_copy(p_hbm.at[i2_v.at[pl.ds(cb, CQ)]], r2x, sem)
        pltpu.async_copy(q_hbm.at[pl.ds(qbase + cb, CQ)], qx, sem)

    def wait4(bset):
        r0x, _, _, qx, sem = bset
        for _k in range(3):
            pltpu.make_async_copy(p_hbm.at[i0_v.at[pl.ds(0, CQ)]], r0x,
                                  sem).wait()
        pltpu.make_async_copy(q_hbm.at[pl.ds(qbase, CQ)], qx, sem).wait()

    def wait_out(ox):
        pltpu.make_async_copy(ox, h1_hbm.at[pl.ds(qbase, CQ)], sem_o).wait()

    def compute(cb, bset, ox):
        r0x, r1x, r2x, qx, _ = bset

        def one_q(qi, _):
            inv = 1.0 / PSCALE
            a0 = jnp.full((L,), w0_v[pl.ds(cb + qi, L)][0] * inv)
            a1 = jnp.full((L,), w1_v[pl.ds(cb + qi, L)][0] * inv)
            a2 = jnp.full((L,), w2_v[pl.ds(cb + qi, L)][0] * inv)
            sixteen = jnp.full((L,), jnp.int32(16))
            lomask = jnp.full((L,), jnp.int32(65535))

            def upk(u32):
                lo = lax.convert_element_type(
                    lax.bitwise_and(u32, lomask), jnp.float32)
                hi = lax.convert_element_type(
                    lax.shift_right_logical(u32, sixteen), jnp.float32)
                return lo, hi

            for v in range(HALF // L):
                s16 = pl.ds(v * L, L)
                p0l, p0h = upk(r0x[qi, s16])
                p1l, p1h = upk(r1x[qi, s16])
                p2l, p2h = upk(r2x[qi, s16])
                slh = pl.ds(HALF + v * L, L)
                acc_l = qx[qi, s16] + a0 * p0l + a1 * p1l + a2 * p2l
                acc_h = qx[qi, slh] + a0 * p0h + a1 * p1h + a2 * p2h
                ox[qi, s16] = acc_l
                ox[qi, slh] = acc_h
                plsc.addupdate(st_v.at[0, s16], acc_l)
                plsc.addupdate(st_v.at[1, s16], acc_l * acc_l)
                plsc.addupdate(st_v.at[0, slh], acc_h)
                plsc.addupdate(st_v.at[1, slh], acc_h * acc_h)
            return _

        lax.fori_loop(0, CQ, one_q, None, unroll=4)
        pltpu.async_copy(ox, h1_hbm.at[pl.ds(qbase + cb, CQ)], sem_o)

    fire(0, bufs[0])

    def pair(h, _):
        g0 = 2 * h
        fire((g0 + 1) * CQ, bufs[1])
        wait4(bufs[0])

        @pl.when(h > 0)
        def _drain_a():
            wait_out(o_a)

        compute(g0 * CQ, bufs[0], o_a)

        @pl.when(g0 + 2 < NCH)
        def _fire_next():
            fire((g0 + 2) * CQ, bufs[0])

        wait4(bufs[1])

        @pl.when(h > 0)
        def _drain_b():
            wait_out(o_b)

        compute((g0 + 1) * CQ, bufs[1], o_b)
        return _

    lax.fori_loop(0, NCH // 2, pair, None, unroll=False)
    wait_out(o_a)
    wait_out(o_b)
    pltpu.sync_copy(st_v, st_hbm.at[wid])


def _sc_interp(p_flat, q_flat, i_all, w_all):
    mesh = plsc.VectorSubcoreMesh(core_axis_name="c", subcore_axis_name="s")
    fn = pl.kernel(
        _sc_body,
        out_type=[
            jax.ShapeDtypeStruct((ROWS2, OUT), jnp.float32),
            jax.ShapeDtypeStruct((NW, 2, OUT), jnp.float32),
        ],
        mesh=mesh,
        scratch_types=[
            pltpu.VMEM((QPW,), jnp.int32),
            pltpu.VMEM((QPW,), jnp.int32),
            pltpu.VMEM((QPW,), jnp.int32),
            pltpu.VMEM((QPW + L,), jnp.float32),
            pltpu.VMEM((QPW + L,), jnp.float32),
            pltpu.VMEM((QPW + L,), jnp.float32),
            pltpu.VMEM((CQ, HALF), jnp.int32),
            pltpu.VMEM((CQ, HALF), jnp.int32),
            pltpu.VMEM((CQ, HALF), jnp.int32),
            pltpu.VMEM((CQ, OUT), jnp.float32),
            pltpu.VMEM((CQ, HALF), jnp.int32),
            pltpu.VMEM((CQ, HALF), jnp.int32),
            pltpu.VMEM((CQ, HALF), jnp.int32),
            pltpu.VMEM((CQ, OUT), jnp.float32),
            pltpu.VMEM((CQ, OUT), jnp.float32),
            pltpu.VMEM((CQ, OUT), jnp.float32),
            pltpu.VMEM((2, OUT), jnp.float32),
            pltpu.SemaphoreType.DMA,
            pltpu.SemaphoreType.DMA,
            pltpu.SemaphoreType.DMA,
        ],
    )
    return fn(p_flat, q_flat, i_all, w_all)


def _mid_body(h1a_ref, h1b_ref, sc_ref, sh_ref, w2t_ref, h2_ref, st_ref):
    r = pl.program_id(0)
    h = jnp.where(r < RT2, h1a_ref[...], h1b_ref[...])
    hn = jnp.maximum(h * sc_ref[...] + sh_ref[...], 0.0)
    h2 = lax.dot_general(hn.astype(jnp.bfloat16),
                         w2t_ref[...].astype(jnp.bfloat16),
                         (((1,), (0,)), ((), ())),
                         preferred_element_type=jnp.float32)
    s1 = jnp.sum(h2, axis=0, keepdims=True)
    s2 = jnp.sum(h2 * h2, axis=0, keepdims=True)
    st_ref[...] = jnp.concatenate([s1, s2], axis=0)[None]
    h2_ref[...] = h2.astype(jnp.bfloat16)


def _mid_layer(h1a, h1b, scale1, shift1, w2t):
    grid_r = ROWS // TN
    return pl.pallas_call(
        _mid_body,
        grid=(grid_r,),
        in_specs=[
            pl.BlockSpec((TN, OUT), lambda r: (jnp.minimum(r, RT2 - 1), 0)),
            pl.BlockSpec((TN, OUT), lambda r: (jnp.maximum(r - RT2, 0), 0)),
            pl.BlockSpec((1, OUT), lambda r: (0, 0)),
            pl.BlockSpec((1, OUT), lambda r: (0, 0)),
            pl.BlockSpec((OUT, OUT), lambda r: (0, 0)),
        ],
        out_specs=[
            pl.BlockSpec((TN, OUT), lambda r: (r, 0)),
            pl.BlockSpec((1, 2, OUT), lambda r: (r, 0, 0)),
        ],
        out_shape=[
            jax.ShapeDtypeStruct((ROWS, OUT), jnp.bfloat16),
            jax.ShapeDtypeStruct((grid_r, 2, OUT), jnp.float32),
        ],
    )(h1a, h1b, scale1, shift1, w2t)


def _out_body(h2_ref, sc_ref, sh_ref, o_ref):
    h = h2_ref[...].astype(jnp.float32)
    y = jnp.maximum(h * sc_ref[...] + sh_ref[...], 0.0)
    o_ref[...] = jnp.transpose(y, (1, 0))[None]


def _final_layer(h2, scale2, shift2):
    return pl.pallas_call(
        _out_body,
        grid=(B, T),
        in_specs=[
            pl.BlockSpec((TN, OUT), lambda b, t: (b * T + t, 0)),
            pl.BlockSpec((1, OUT), lambda b, t: (0, 0)),
            pl.BlockSpec((1, OUT), lambda b, t: (0, 0)),
        ],
        out_specs=pl.BlockSpec((1, OUT, TN), lambda b, t: (b, 0, t)),
        out_shape=jax.ShapeDtypeStruct((B, OUT, N), jnp.float32),
    )(h2, scale2, shift2)


def _fold_stats(sums, sumsq, g, bb):
    mean = sums / float(ROWS)
    var = sumsq / float(ROWS) - mean * mean
    inv = g / jnp.sqrt(var + EPS)
    scale = inv.reshape(1, OUT)
    shift = (bb - mean * inv).reshape(1, OUT)
    return scale, shift


@jax.jit
def kernel(x, xyz, sub_x, sub_xyz, W1, g1, b1, W2, g2, b2):
    sub_xyzt = jnp.transpose(sub_xyz, (0, 2, 1))
    w1a = W1[:, :C_DST]
    w1b = W1[:, C_DST:]

    p_flat = _project_sub(sub_x, w1b)
    qa_flat, ia, wa = _knn_and_skip(xyz, sub_xyzt, x, w1a, 0)
    h1a, st1a = _sc_interp(p_flat, qa_flat, ia, wa)
    qb_flat, ib, wb = _knn_and_skip(xyz, sub_xyzt, x, w1a, HB)
    h1b, st1b = _sc_interp(p_flat, qb_flat, ib, wb)

    s1 = jnp.sum(st1a, axis=0) + jnp.sum(st1b, axis=0)
    scale1, shift1 = _fold_stats(s1[0], s1[1], g1, b1)

    h2, st2 = _mid_layer(h1a, h1b, scale1, shift1, jnp.transpose(W2))
    s2 = jnp.sum(st2, axis=0)
    scale2, shift2 = _fold_stats(s2[0], s2[1], g2, b2)

    return _final_layer(h2, scale2, shift2)
